# Initial kernel scaffold; baseline (speedup 1.0000x reference)
import jax
import jax.numpy as jnp
from jax.experimental import pallas as pl


def kernel(sample, entity_embedding, relation_embedding):
    raise NotImplementedError("write your pallas kernel here")


# speedup vs baseline: 1.8443x; 1.8443x over previous
import functools

import jax
import jax.numpy as jnp
from jax import lax
from jax.experimental import pallas as pl
from jax.experimental.pallas import tpu as pltpu
from jax.experimental.pallas import tpu_sc as plsc

GAMMA = 12.0
D = 128
LANES = 16
NUM_WORKERS = 32


@functools.partial(jax.jit, static_argnums=(3,))
def _sc_score(sample_t, entity_embedding, relation_embedding, batch):
    bpw = batch // NUM_WORKERS
    ch = 128
    nch = bpw // ch
    mesh = plsc.VectorSubcoreMesh(core_axis_name="c", subcore_axis_name="s")

    @functools.partial(
        pl.kernel,
        mesh=mesh,
        out_type=jax.ShapeDtypeStruct((batch,), jnp.float32),
        scratch_types=[
            pltpu.VMEM((bpw,), jnp.int32),
            pltpu.VMEM((bpw,), jnp.int32),
            pltpu.VMEM((bpw,), jnp.int32),
            pltpu.VMEM((ch, D), jnp.float32),
            pltpu.VMEM((ch, D), jnp.float32),
            pltpu.VMEM((ch, D), jnp.float32),
            pltpu.VMEM((bpw,), jnp.float32),
            pltpu.SemaphoreType.DMA,
        ],
    )
    def body(hs_hbm, rs_hbm, ts_hbm, ent_hbm, rel_hbm, out_hbm,
             hs_v, rs_v, ts_v, hbuf, rbuf, tbuf, out_v, sem):
        wid = lax.axis_index("s") * 2 + lax.axis_index("c")
        base = wid * bpw
        pltpu.sync_copy(hs_hbm.at[pl.ds(base, bpw)], hs_v)
        pltpu.sync_copy(rs_hbm.at[pl.ds(base, bpw)], rs_v)
        pltpu.sync_copy(ts_hbm.at[pl.ds(base, bpw)], ts_v)

        def chunk(c, carry):
            off = c * ch
            cp_h = pltpu.async_copy(ent_hbm.at[hs_v.at[pl.ds(off, ch)]], hbuf, sem)
            cp_r = pltpu.async_copy(rel_hbm.at[rs_v.at[pl.ds(off, ch)]], rbuf, sem)
            cp_t = pltpu.async_copy(ent_hbm.at[ts_v.at[pl.ds(off, ch)]], tbuf, sem)
            cp_h.wait()
            cp_r.wait()
            cp_t.wait()

            lane = lax.iota(jnp.int32, LANES)

            def lanesum(v):
                for sh in (1, 2, 4, 8):
                    v = v + v.at[lane ^ sh].get(mode="promise_in_bounds")
                return v

            def group(g, carry2):
                svec = jnp.full((LANES,), GAMMA, jnp.float32)
                for j in range(LANES):
                    i = g * LANES + j
                    acc = jnp.zeros((LANES,), jnp.float32)
                    for k in range(D // LANES):
                        sl = pl.ds(k * LANES, LANES)
                        acc = acc + jnp.abs(hbuf[i, sl] + rbuf[i, sl] - tbuf[i, sl])
                    svec = jnp.where(lane == j, svec - lanesum(acc), svec)
                out_v[pl.ds(off + g * LANES, LANES)] = svec
                return carry2

            lax.fori_loop(0, ch // LANES, group, 0)
            return carry

        lax.fori_loop(0, nch, chunk, 0)
        pltpu.sync_copy(out_v, out_hbm.at[pl.ds(base, bpw)])

    return body(sample_t[0], sample_t[1], sample_t[2],
                entity_embedding, relation_embedding)


def kernel(sample, entity_embedding, relation_embedding):
    batch = sample.shape[0]
    sample_t = sample.T
    score = _sc_score(sample_t, entity_embedding, relation_embedding, batch)
    return score.reshape(batch, 1)


# speedup vs baseline: 1.9397x; 1.0517x over previous
import functools

import jax
import jax.numpy as jnp
from jax import lax
from jax.experimental import pallas as pl
from jax.experimental.pallas import tpu as pltpu
from jax.experimental.pallas import tpu_sc as plsc

GAMMA = 12.0
D = 128
LANES = 16
NUM_WORKERS = 32
CHUNK = 128


@functools.partial(jax.jit, static_argnums=(3,))
def _sc_score(sample_t, entity_embedding, relation_embedding, batch):
    bpw = batch // NUM_WORKERS
    nch = bpw // CHUNK
    mesh = plsc.VectorSubcoreMesh(core_axis_name="c", subcore_axis_name="s")

    @functools.partial(
        pl.kernel,
        mesh=mesh,
        out_type=jax.ShapeDtypeStruct((batch,), jnp.float32),
        scratch_types=[
            pltpu.VMEM((bpw,), jnp.int32),
            pltpu.VMEM((bpw,), jnp.int32),
            pltpu.VMEM((bpw,), jnp.int32),
            pltpu.VMEM((CHUNK, D), jnp.float32),
            pltpu.VMEM((CHUNK, D), jnp.float32),
            pltpu.VMEM((CHUNK, D), jnp.float32),
            pltpu.VMEM((CHUNK, D), jnp.float32),
            pltpu.VMEM((CHUNK, D), jnp.float32),
            pltpu.VMEM((CHUNK, D), jnp.float32),
            pltpu.VMEM((bpw,), jnp.float32),
            pltpu.SemaphoreType.DMA,
            pltpu.SemaphoreType.DMA,
        ],
    )
    def body(hs_hbm, rs_hbm, ts_hbm, ent_hbm, rel_hbm, out_hbm,
             hs_v, rs_v, ts_v, hb0, rb0, tb0, hb1, rb1, tb1, out_v,
             sem0, sem1):
        wid = lax.axis_index("s") * 2 + lax.axis_index("c")
        base = wid * bpw
        pltpu.sync_copy(hs_hbm.at[pl.ds(base, bpw)], hs_v)
        pltpu.sync_copy(rs_hbm.at[pl.ds(base, bpw)], rs_v)
        pltpu.sync_copy(ts_hbm.at[pl.ds(base, bpw)], ts_v)

        bufs = ((hb0, rb0, tb0), (hb1, rb1, tb1))
        sems = (sem0, sem1)
        lane = lax.iota(jnp.int32, LANES)

        def start(c):
            hb, rb, tb = bufs[c % 2]
            sem = sems[c % 2]
            off = c * CHUNK
            return (
                pltpu.async_copy(ent_hbm.at[hs_v.at[pl.ds(off, CHUNK)]], hb, sem),
                pltpu.async_copy(rel_hbm.at[rs_v.at[pl.ds(off, CHUNK)]], rb, sem),
                pltpu.async_copy(ent_hbm.at[ts_v.at[pl.ds(off, CHUNK)]], tb, sem),
            )

        def lanesum(v):
            for sh in (1, 2, 4, 8):
                v = v + v.at[lane ^ sh].get(mode="promise_in_bounds")
            return v

        def compute(c):
            hb, rb, tb = bufs[c % 2]
            off = c * CHUNK

            def group(g, carry):
                svec = jnp.full((LANES,), GAMMA, jnp.float32)
                for j in range(LANES):
                    i = g * LANES + j
                    acc = jnp.zeros((LANES,), jnp.float32)
                    for k in range(D // LANES):
                        sl = pl.ds(k * LANES, LANES)
                        acc = acc + jnp.abs(hb[i, sl] + rb[i, sl] - tb[i, sl])
                    svec = jnp.where(lane == j, svec - lanesum(acc), svec)
                out_v[pl.ds(off + g * LANES, LANES)] = svec
                return carry

            lax.fori_loop(0, CHUNK // LANES, group, 0)

        cps = start(0)
        for c in range(nch):
            nxt = start(c + 1) if c + 1 < nch else None
            for cp in cps:
                cp.wait()
            compute(c)
            cps = nxt

        pltpu.sync_copy(out_v, out_hbm.at[pl.ds(base, bpw)])

    return body(sample_t[0], sample_t[1], sample_t[2],
                entity_embedding, relation_embedding)


def kernel(sample, entity_embedding, relation_embedding):
    batch = sample.shape[0]
    sample_t = sample.T
    score = _sc_score(sample_t, entity_embedding, relation_embedding, batch)
    return score.reshape(batch, 1)
